# Initial kernel scaffold; baseline (speedup 1.0000x reference)
import jax
import jax.numpy as jnp
from jax.experimental import pallas as pl


def kernel(data, index, weights):
    raise NotImplementedError("write your pallas kernel here")


# speedup vs baseline: 5.8980x; 5.8980x over previous
import functools

import jax
import jax.numpy as jnp
from jax.experimental import pallas as pl
from jax.experimental.pallas import tpu as pltpu

N_NODES = 10000
N_EDGES = 160000
D_FEAT = 128
MAX_VAL = 1.0
EPS = 1e-7

B = 640
NB = 80
C = N_EDGES // B
J = N_NODES // NB
T_MAX = C + J + 1


def _laf_body(meta_ref, w_ref, data_ref, idx_ref, out_ref, acc_ref):
    t = pl.program_id(0)
    j = meta_ref[1, t]
    first = meta_ref[2, t]
    last = meta_ref[3, t]

    @pl.when(first == 1)
    def _zero():
        acc_ref[...] = jnp.zeros_like(acc_ref)

    data = data_ref[...]
    x = jnp.clip(data, EPS, 1.0 - EPS)
    xm = jnp.clip(MAX_VAL - x, EPS, 1.0 - EPS)
    lx = jnp.log(x)
    lxm = jnp.log(xm)
    t4 = jnp.concatenate(
        [jnp.exp(w_ref[2, 0] * lx), jnp.exp(w_ref[5, 0] * lxm),
         jnp.exp(w_ref[8, 0] * lx), jnp.exp(w_ref[11, 0] * lxm)],
        axis=1)

    rows = j * NB + jax.lax.broadcasted_iota(jnp.int32, (NB, B), 0)
    idx = idx_ref[...].reshape(1, B)
    p = (rows == idx).astype(jnp.float32)
    acc_ref[...] += jnp.dot(p, t4, preferred_element_type=jnp.float32)

    @pl.when(last == 1)
    def _finalize():
        s = acc_ref[...]
        ls = jnp.log(jnp.maximum(s, EPS))
        l1 = ls[:, 0:128]
        l2 = ls[:, 128:256]
        l3 = ls[:, 256:384]
        l4 = ls[:, 384:512]
        num = (w_ref[0, 0] * jnp.exp(w_ref[1, 0] * l1)
               + w_ref[3, 0] * jnp.exp(w_ref[4, 0] * l2))
        den = (w_ref[6, 0] * jnp.exp(w_ref[7, 0] * l3)
               + w_ref[9, 0] * jnp.exp(w_ref[10, 0] * l4))
        mult = 2.0 * jnp.clip(jnp.sign(den), 0.0, None) - 1.0
        den = jnp.where(jnp.abs(den) < EPS, mult * EPS, den)
        out_ref[...] = num / den


@functools.partial(jax.jit, static_argnames=("interpret",))
def _laf(data, index, weights, interpret=False):
    bounds = jnp.arange(0, N_NODES + 1, NB, dtype=jnp.int32)
    row_start = jnp.searchsorted(index, bounds, side="left").astype(jnp.int32)
    s0 = row_start[:-1]
    s1 = row_start[1:]
    clo = jnp.clip(s0 // B, 0, C - 1)
    chi = jnp.clip((jnp.maximum(s1, s0 + 1) - 1) // B, 0, C - 1)
    chi = jnp.maximum(chi, clo)
    count = chi - clo + 1
    tstart = jnp.concatenate(
        [jnp.zeros((1,), jnp.int32), jnp.cumsum(count, dtype=jnp.int32)])
    t = jnp.arange(T_MAX, dtype=jnp.int32)
    j_of_t = jnp.clip(
        jnp.searchsorted(tstart, t, side="right").astype(jnp.int32) - 1,
        0, J - 1)
    c_of_t = jnp.clip(clo[j_of_t] + (t - tstart[j_of_t]), 0, C - 1)
    first = (t == tstart[j_of_t]).astype(jnp.int32)
    last = (t == tstart[j_of_t + 1] - 1).astype(jnp.int32)
    meta = jnp.stack([c_of_t, j_of_t, first, last])

    idx3 = index.reshape(C, 1, B)

    grid_spec = pltpu.PrefetchScalarGridSpec(
        num_scalar_prefetch=2,
        grid=(T_MAX,),
        in_specs=[
            pl.BlockSpec((B, D_FEAT), lambda t, m, w: (m[0, t], 0)),
            pl.BlockSpec((1, 1, B), lambda t, m, w: (m[0, t], 0, 0)),
        ],
        out_specs=pl.BlockSpec((NB, D_FEAT), lambda t, m, w: (m[1, t], 0)),
        scratch_shapes=[pltpu.VMEM((NB, 4 * D_FEAT), jnp.float32)],
    )
    out = pl.pallas_call(
        _laf_body,
        grid_spec=grid_spec,
        out_shape=jax.ShapeDtypeStruct((N_NODES, D_FEAT), jnp.float32),
        interpret=interpret,
    )(meta, weights, data, idx3)
    return out[:, :, None]


def kernel(data, index, weights):
    return _laf(data, index, weights)
